# baseline (speedup 1.0000x reference)
import functools

import numpy as np
import jax
import jax.numpy as jnp
from jax import lax
from jax.experimental import pallas as pl
from jax.experimental.pallas import tpu as pltpu
from jax.experimental.pallas import tpu_sc as plsc

N = 10000
INC = 320000
D = 128
K = 10
TEMP = 30.0

NC, NS = 2, 16
NW = NC * NS
PER_W = INC // NW
CHUNK = 80
NCH = PER_W // CHUNK
NP = 10240
RPT = NP // NS

f32 = jnp.float32


@functools.cache
def _sc_mesh():
    return plsc.VectorSubcoreMesh(core_axis_name="c", subcore_axis_name="s",
                                  num_cores=NC, num_subcores=NS)

_rng = np.random.RandomState(0)
_I0 = int(_rng.randint(N))
_US = [float(_rng.random_sample()) for _ in range(K - 1)]

NB = 50
RB2 = 200


@functools.cache
def _sc_seg_sum_kernel(width):
    def body(table, gidx, sidx, zeros, out,
             gi0, gi1, gi2, si0, si1, si2, r0, r1, r2,
             g0, g1, g2, s0, s1, s2, acc):
        c = lax.axis_index("c")
        s = lax.axis_index("s")
        wid = s * NC + c
        slot = [(gi0, si0, r0, g0, s0), (gi1, si1, r1, g1, s1),
                (gi2, si2, r2, g2, s2)]
        pltpu.sync_copy(zeros, acc.at[pl.ds(s * RPT, RPT)])
        plsc.subcore_barrier()

        def fire(j, p, wait_scatter):
            gi, si, rows, gsem, ssem = slot[p]
            if wait_scatter:
                pltpu.make_async_copy(rows, acc.at[si], ssem).wait()
            pltpu.sync_copy(gidx.at[wid, j], gi)
            pltpu.sync_copy(sidx.at[wid, j], si)
            pltpu.async_copy(table.at[gi], rows, gsem)

        def scat(j, p):
            gi, si, rows, gsem, ssem = slot[p]
            pltpu.make_async_copy(table.at[gi], rows, gsem).wait()
            pltpu.async_copy(rows, acc.at[si], ssem, add=True)

        fire(0, 0, False)
        fire(1, 1, False)

        def step(t, carry):
            for u in range(3):
                j = 3 * t + 2 + u
                p = (2 + u) % 3
                scat(j - 2, (p + 1) % 3)
                if u == 0:
                    @pl.when(t > 0)
                    def _():
                        gi, si, rows, gsem, ssem = slot[p]
                        pltpu.make_async_copy(rows, acc.at[si], ssem).wait()
                    gi, si, rows, gsem, ssem = slot[p]
                    pltpu.sync_copy(gidx.at[wid, j], gi)
                    pltpu.sync_copy(sidx.at[wid, j], si)
                    pltpu.async_copy(table.at[gi], rows, gsem)
                else:
                    fire(j, p, True)
            return carry

        lax.fori_loop(0, (NCH - 2) // 3, step, 0)
        scat(NCH - 2, (NCH - 2) % 3)
        scat(NCH - 1, (NCH - 1) % 3)
        for p in range(3):
            gi, si, rows, gsem, ssem = slot[p]
            pltpu.make_async_copy(rows, acc.at[si], ssem).wait()
        plsc.subcore_barrier()
        pltpu.sync_copy(acc.at[pl.ds(s * RPT, RPT)],
                        out.at[c, pl.ds(s * RPT, RPT)])

    idx_t = pltpu.VMEM((CHUNK,), jnp.int32)
    rows_t = pltpu.VMEM((CHUNK, width), f32)
    return pl.kernel(
        body,
        out_type=jax.ShapeDtypeStruct((NC, NP, width), f32),
        mesh=_sc_mesh(),
        scratch_types=[
            idx_t, idx_t, idx_t, idx_t, idx_t, idx_t,
            rows_t, rows_t, rows_t,
            pltpu.SemaphoreType.DMA, pltpu.SemaphoreType.DMA,
            pltpu.SemaphoreType.DMA, pltpu.SemaphoreType.DMA,
            pltpu.SemaphoreType.DMA, pltpu.SemaphoreType.DMA,
            pltpu.VMEM_SHARED((NP, width), f32),
        ],
    )


def _sc_seg_sum(table, gidx, sidx, zeros):
    return _sc_seg_sum_kernel(D)(table, gidx, sidx, zeros)


@functools.cache
def _sc_count_kernel():
    def body(sidx, ones_h, zeros, out, si_v, ones_v, acc):
        c = lax.axis_index("c")
        s = lax.axis_index("s")
        wid = s * NC + c
        pltpu.sync_copy(zeros, acc.at[pl.ds(s * RPT, RPT)])
        pltpu.sync_copy(ones_h, ones_v)
        plsc.subcore_barrier()

        def step(j, carry):
            pltpu.sync_copy(sidx.at[wid, j], si_v)
            pltpu.sync_copy(ones_v, acc.at[si_v], add=True)
            return carry

        lax.fori_loop(0, NCH, step, 0)
        plsc.subcore_barrier()
        pltpu.sync_copy(acc.at[pl.ds(s * RPT, RPT)],
                        out.at[c, pl.ds(s * RPT, RPT)])

    return pl.kernel(
        body,
        out_type=jax.ShapeDtypeStruct((NC, NP, D), f32),
        mesh=_sc_mesh(),
        scratch_types=[
            pltpu.VMEM((CHUNK,), jnp.int32),
            pltpu.VMEM((CHUNK, D), f32),
            pltpu.VMEM_SHARED((NP, D), f32),
        ],
    )


def _sc_count(sidx, ones_h, zeros):
    return _sc_count_kernel()(sidx, ones_h, zeros)


_RB = 2000


def _mm_bias_body(x_ref, w_ref, b_ref, o_ref):
    o_ref[...] = jnp.dot(x_ref[...], w_ref[...],
                         preferred_element_type=f32) + b_ref[...]


def _mm_bias(x, W, b):
    return pl.pallas_call(
        _mm_bias_body,
        grid=(N // _RB,),
        in_specs=[pl.BlockSpec((_RB, D), lambda i: (i, 0)),
                  pl.BlockSpec((D, D), lambda i: (0, 0)),
                  pl.BlockSpec((1, D), lambda i: (0, 0))],
        out_specs=pl.BlockSpec((_RB, D), lambda i: (i, 0)),
        out_shape=jax.ShapeDtypeStruct((N, D), f32),
    )(x, W, b.reshape(1, D))


def _seg_mean(p, cnt):
    def body(p_ref, c_ref, o_ref):
        c = c_ref[0, :, 0:1] + c_ref[1, :, 0:1]
        o_ref[...] = (p_ref[0] + p_ref[1]) / jnp.maximum(c, 1.0)

    return pl.pallas_call(
        body,
        grid=(N // _RB,),
        in_specs=[pl.BlockSpec((NC, _RB, D), lambda i: (0, i, 0)),
                  pl.BlockSpec((NC, _RB, D), lambda i: (0, i, 0))],
        out_specs=pl.BlockSpec((_RB, D), lambda i: (i, 0)),
        out_shape=jax.ShapeDtypeStruct((N, D), f32),
    )(p, cnt)


def _seg_mean_relu_mm(p, cnt, W, b):
    def body(p_ref, c_ref, w_ref, b_ref, o_ref):
        c = c_ref[0, :, 0:1] + c_ref[1, :, 0:1]
        h = jnp.maximum((p_ref[0] + p_ref[1]) / jnp.maximum(c, 1.0), 0.0)
        o_ref[...] = jnp.dot(h, w_ref[...],
                             preferred_element_type=f32) + b_ref[...]

    return pl.pallas_call(
        body,
        grid=(N // _RB,),
        in_specs=[pl.BlockSpec((NC, _RB, D), lambda i: (0, i, 0)),
                  pl.BlockSpec((NC, _RB, D), lambda i: (0, i, 0)),
                  pl.BlockSpec((D, D), lambda i: (0, 0)),
                  pl.BlockSpec((1, D), lambda i: (0, 0))],
        out_specs=pl.BlockSpec((_RB, D), lambda i: (i, 0)),
        out_shape=jax.ShapeDtypeStruct((N, D), f32),
    )(p, cnt, W, b.reshape(1, D))


def _finalize(p, cnt):
    def body(p_ref, c_ref, e_ref, d_ref):
        c = c_ref[0, :, 0:1] + c_ref[1, :, 0:1]
        emb = (p_ref[0] + p_ref[1]) / jnp.maximum(c, 1.0)
        e_ref[...] = emb
        nrm = jnp.sqrt(jnp.sum(emb * emb, axis=1, keepdims=True))
        d_ref[...] = emb / nrm

    return pl.pallas_call(
        body,
        grid=(N // _RB,),
        in_specs=[pl.BlockSpec((NC, _RB, D), lambda i: (0, i, 0)),
                  pl.BlockSpec((NC, _RB, D), lambda i: (0, i, 0))],
        out_specs=[pl.BlockSpec((_RB, D), lambda i: (i, 0)),
                   pl.BlockSpec((_RB, D), lambda i: (i, 0))],
        out_shape=[jax.ShapeDtypeStruct((N, D), f32),
                   jax.ShapeDtypeStruct((N, D), f32)],
    )(p, cnt)


def _kmeanspp_body(x_ref, mu_ref, w_scr):
    xx = x_ref[...]
    c0 = xx[_I0:_I0 + 1, :]
    centers = [c0]
    closest = jnp.sum((xx - c0) ** 2, axis=1, keepdims=True)

    rowb = lax.broadcasted_iota(jnp.int32, (NB, N), 0)
    colb = lax.broadcasted_iota(jnp.int32, (NB, N), 1)
    B0 = (colb // RB2 == rowb).astype(f32)
    r50a = lax.broadcasted_iota(jnp.int32, (NB, NB), 0)
    r50b = lax.broadcasted_iota(jnp.int32, (NB, NB), 1)
    T50 = (r50b <= r50a).astype(f32)
    ra = lax.broadcasted_iota(jnp.int32, (RB2, RB2), 0)
    rb = lax.broadcasted_iota(jnp.int32, (RB2, RB2), 1)
    TR = (rb <= ra).astype(f32)
    oh_col = lax.broadcasted_iota(jnp.int32, (1, N), 1)

    for i in range(1, K):
        u = _US[i - 1]
        total = jnp.sum(closest)
        pos = total > 0.0
        w = jnp.where(pos, closest, 1.0)
        t = u * jnp.where(pos, total, float(N))
        sb = jnp.dot(B0, w, preferred_element_type=f32)
        bp = jnp.dot(T50, sb, preferred_element_type=f32)
        ble = bp <= t
        b_star = jnp.minimum(jnp.sum(ble.astype(jnp.int32)), NB - 1)
        off = jnp.sum(jnp.where(ble, sb, 0.0))
        w_scr[...] = w
        blk = w_scr[pl.ds(b_star * RB2, RB2), :]
        wp = jnp.dot(TR, blk, preferred_element_type=f32)
        cnt_in = jnp.sum((off + wp <= t).astype(jnp.int32))
        idx = jnp.minimum(b_star * RB2 + cnt_in, N - 1)
        oh = (oh_col == idx).astype(f32)
        ci = jnp.dot(oh, xx, preferred_element_type=f32)
        centers.append(ci)
        closest = jnp.minimum(closest,
                              jnp.sum((xx - ci) ** 2, axis=1, keepdims=True))
    mu_ref[...] = jnp.concatenate(centers, axis=0)


def _kmeanspp(data):
    return pl.pallas_call(
        _kmeanspp_body,
        out_shape=jax.ShapeDtypeStruct((K, D), f32),
        scratch_shapes=[pltpu.VMEM((N, 1), f32)],
    )(data)


def _cluster_body(ni_ref, x_ref, mu0_ref, mu_out, r_out, d_out):
    data = x_ref[...]
    ones_col = jnp.ones((N, 1), f32)

    def softmax_r(dist):
        sc = TEMP * dist
        m = jnp.max(sc, axis=1, keepdims=True)
        e = jnp.exp(sc - m)
        return e / jnp.sum(e, axis=1, keepdims=True)

    def body(_, mu):
        dist = lax.dot_general(data, mu, (((1,), (1,)), ((), ())),
                               preferred_element_type=f32)
        r = softmax_r(dist)
        cm = lax.dot_general(r, data, (((0,), (0,)), ((), ())),
                             preferred_element_type=f32)
        cr = lax.dot_general(r, ones_col, (((0,), (0,)), ((), ())),
                             preferred_element_type=f32)
        return cm / cr

    mu = lax.fori_loop(0, ni_ref[0], body, mu0_ref[...])
    dist = lax.dot_general(data, mu, (((1,), (1,)), ((), ())),
                           preferred_element_type=f32)
    r = softmax_r(dist)
    mu_out[...] = mu
    r_out[...] = r
    d_out[...] = dist


def _cluster(ni, data, mu0):
    return pl.pallas_call(
        _cluster_body,
        in_specs=[pl.BlockSpec(memory_space=pltpu.SMEM),
                  pl.BlockSpec((N, D), lambda: (0, 0)),
                  pl.BlockSpec((K, D), lambda: (0, 0))],
        out_specs=[pl.BlockSpec((K, D), lambda: (0, 0)),
                   pl.BlockSpec((N, K), lambda: (0, 0)),
                   pl.BlockSpec((N, K), lambda: (0, 0))],
        out_shape=[jax.ShapeDtypeStruct((K, D), f32),
                   jax.ShapeDtypeStruct((N, K), f32),
                   jax.ShapeDtypeStruct((N, K), f32)],
    )(ni, data, mu0)


def kernel(x, edge_index, W1, b1, W2, b2, num_iter):
    v_idx = edge_index[0]
    e_idx = edge_index[1]
    vi3 = v_idx.reshape(NW, NCH, CHUNK)
    ei3 = e_idx.reshape(NW, NCH, CHUNK)
    zrow = jnp.zeros((RPT, D), f32)
    ocnt = jnp.ones((CHUNK, D), f32)

    vcntp = _sc_count(vi3, ocnt, zrow)
    ecntp = _sc_count(ei3, ocnt, zrow)

    h1 = _mm_bias(x, W1, b1)
    pa = _sc_seg_sum(h1, vi3, ei3, zrow)
    he1 = _seg_mean(pa, ecntp)
    pb = _sc_seg_sum(he1, ei3, vi3, zrow)
    h2 = _seg_mean_relu_mm(pb, vcntp, W2, b2)
    pa2 = _sc_seg_sum(h2, vi3, ei3, zrow)
    he2 = _seg_mean(pa2, ecntp)
    pb2 = _sc_seg_sum(he2, ei3, vi3, zrow)
    embeds, data = _finalize(pb2, vcntp)

    mu0 = _kmeanspp(data)
    ni = jnp.asarray(num_iter, jnp.int32).reshape((1,))
    mu, r, dist = _cluster(ni, data, mu0)
    return mu, r, embeds, dist


# speedup vs baseline: 1.2103x; 1.2103x over previous
import functools

import numpy as np
import jax
import jax.numpy as jnp
from jax import lax
from jax.experimental import pallas as pl
from jax.experimental.pallas import tpu as pltpu
from jax.experimental.pallas import tpu_sc as plsc

N = 10000
INC = 320000
D = 128
K = 10
TEMP = 30.0

NC, NS = 2, 16
NW = NC * NS
PER_W = INC // NW
CHUNK = 125
NCH = PER_W // CHUNK
NP = 10112
RPT = NP // NS

f32 = jnp.float32


@functools.cache
def _sc_mesh():
    return plsc.VectorSubcoreMesh(core_axis_name="c", subcore_axis_name="s",
                                  num_cores=NC, num_subcores=NS)

_rng = np.random.RandomState(0)
_I0 = int(_rng.randint(N))
_US = [float(_rng.random_sample()) for _ in range(K - 1)]

NB = 50
RB2 = 200


@functools.cache
def _sc_seg_sum_kernel(width):
    def body(table, gidx, sidx, zeros, out,
             gi0, gi1, gi2, si0, si1, si2, r0, r1, r2,
             g0, g1, g2, s0, s1, s2, acc):
        c = lax.axis_index("c")
        s = lax.axis_index("s")
        wid = s * NC + c
        slot = [(gi0, si0, r0, g0, s0), (gi1, si1, r1, g1, s1),
                (gi2, si2, r2, g2, s2)]
        pltpu.sync_copy(zeros, acc.at[pl.ds(s * RPT, RPT)])
        plsc.subcore_barrier()

        def fire(j, p, wait_scatter):
            gi, si, rows, gsem, ssem = slot[p]
            if wait_scatter:
                pltpu.make_async_copy(rows, acc.at[si], ssem).wait()
            pltpu.sync_copy(gidx.at[wid, j], gi)
            pltpu.sync_copy(sidx.at[wid, j], si)
            pltpu.async_copy(table.at[gi], rows, gsem)

        def scat(j, p):
            gi, si, rows, gsem, ssem = slot[p]
            pltpu.make_async_copy(table.at[gi], rows, gsem).wait()
            pltpu.async_copy(rows, acc.at[si], ssem, add=True)

        fire(0, 0, False)
        fire(1, 1, False)

        def step(t, carry):
            for u in range(3):
                j = 3 * t + 2 + u
                p = (2 + u) % 3
                scat(j - 2, (p + 1) % 3)
                if u == 0:
                    @pl.when(t > 0)
                    def _():
                        gi, si, rows, gsem, ssem = slot[p]
                        pltpu.make_async_copy(rows, acc.at[si], ssem).wait()
                    gi, si, rows, gsem, ssem = slot[p]
                    pltpu.sync_copy(gidx.at[wid, j], gi)
                    pltpu.sync_copy(sidx.at[wid, j], si)
                    pltpu.async_copy(table.at[gi], rows, gsem)
                else:
                    fire(j, p, True)
            return carry

        lax.fori_loop(0, (NCH - 2) // 3, step, 0)
        scat(NCH - 2, (NCH - 2) % 3)
        scat(NCH - 1, (NCH - 1) % 3)
        for p in range(3):
            gi, si, rows, gsem, ssem = slot[p]
            pltpu.make_async_copy(rows, acc.at[si], ssem).wait()
        plsc.subcore_barrier()
        pltpu.sync_copy(acc.at[pl.ds(s * RPT, RPT)],
                        out.at[c, pl.ds(s * RPT, RPT)])

    idx_t = pltpu.VMEM((CHUNK,), jnp.int32)
    rows_t = pltpu.VMEM((CHUNK, width), f32)
    return pl.kernel(
        body,
        out_type=jax.ShapeDtypeStruct((NC, NP, width), f32),
        mesh=_sc_mesh(),
        scratch_types=[
            idx_t, idx_t, idx_t, idx_t, idx_t, idx_t,
            rows_t, rows_t, rows_t,
            pltpu.SemaphoreType.DMA, pltpu.SemaphoreType.DMA,
            pltpu.SemaphoreType.DMA, pltpu.SemaphoreType.DMA,
            pltpu.SemaphoreType.DMA, pltpu.SemaphoreType.DMA,
            pltpu.VMEM_SHARED((NP, width), f32),
        ],
    )


def _sc_seg_sum(table, gidx, sidx, zeros):
    return _sc_seg_sum_kernel(D)(table, gidx, sidx, zeros)


@functools.cache
def _sc_count_kernel():
    def body(sidx, ones_h, zeros, out, si_v, ones_v, acc):
        c = lax.axis_index("c")
        s = lax.axis_index("s")
        wid = s * NC + c
        pltpu.sync_copy(zeros, acc.at[pl.ds(s * RPT, RPT)])
        pltpu.sync_copy(ones_h, ones_v)
        plsc.subcore_barrier()

        def step(j, carry):
            pltpu.sync_copy(sidx.at[wid, j], si_v)
            pltpu.sync_copy(ones_v, acc.at[si_v], add=True)
            return carry

        lax.fori_loop(0, NCH, step, 0)
        plsc.subcore_barrier()
        pltpu.sync_copy(acc.at[pl.ds(s * RPT, RPT)],
                        out.at[c, pl.ds(s * RPT, RPT)])

    return pl.kernel(
        body,
        out_type=jax.ShapeDtypeStruct((NC, NP, D), f32),
        mesh=_sc_mesh(),
        scratch_types=[
            pltpu.VMEM((CHUNK,), jnp.int32),
            pltpu.VMEM((CHUNK, D), f32),
            pltpu.VMEM_SHARED((NP, D), f32),
        ],
    )


def _sc_count(sidx, ones_h, zeros):
    return _sc_count_kernel()(sidx, ones_h, zeros)


_RB = 2000


def _mm_bias_body(x_ref, w_ref, b_ref, o_ref):
    o_ref[...] = jnp.dot(x_ref[...], w_ref[...],
                         preferred_element_type=f32) + b_ref[...]


def _mm_bias(x, W, b):
    return pl.pallas_call(
        _mm_bias_body,
        grid=(N // _RB,),
        in_specs=[pl.BlockSpec((_RB, D), lambda i: (i, 0)),
                  pl.BlockSpec((D, D), lambda i: (0, 0)),
                  pl.BlockSpec((1, D), lambda i: (0, 0))],
        out_specs=pl.BlockSpec((_RB, D), lambda i: (i, 0)),
        out_shape=jax.ShapeDtypeStruct((N, D), f32),
    )(x, W, b.reshape(1, D))


def _seg_mean(p, cnt):
    def body(p_ref, c_ref, o_ref):
        c = c_ref[0, :, 0:1] + c_ref[1, :, 0:1]
        o_ref[...] = (p_ref[0] + p_ref[1]) / jnp.maximum(c, 1.0)

    return pl.pallas_call(
        body,
        grid=(N // _RB,),
        in_specs=[pl.BlockSpec((NC, _RB, D), lambda i: (0, i, 0)),
                  pl.BlockSpec((NC, _RB, D), lambda i: (0, i, 0))],
        out_specs=pl.BlockSpec((_RB, D), lambda i: (i, 0)),
        out_shape=jax.ShapeDtypeStruct((N, D), f32),
    )(p, cnt)


def _seg_mean_relu_mm(p, cnt, W, b):
    def body(p_ref, c_ref, w_ref, b_ref, o_ref):
        c = c_ref[0, :, 0:1] + c_ref[1, :, 0:1]
        h = jnp.maximum((p_ref[0] + p_ref[1]) / jnp.maximum(c, 1.0), 0.0)
        o_ref[...] = jnp.dot(h, w_ref[...],
                             preferred_element_type=f32) + b_ref[...]

    return pl.pallas_call(
        body,
        grid=(N // _RB,),
        in_specs=[pl.BlockSpec((NC, _RB, D), lambda i: (0, i, 0)),
                  pl.BlockSpec((NC, _RB, D), lambda i: (0, i, 0)),
                  pl.BlockSpec((D, D), lambda i: (0, 0)),
                  pl.BlockSpec((1, D), lambda i: (0, 0))],
        out_specs=pl.BlockSpec((_RB, D), lambda i: (i, 0)),
        out_shape=jax.ShapeDtypeStruct((N, D), f32),
    )(p, cnt, W, b.reshape(1, D))


def _finalize(p, cnt):
    def body(p_ref, c_ref, e_ref, d_ref):
        c = c_ref[0, :, 0:1] + c_ref[1, :, 0:1]
        emb = (p_ref[0] + p_ref[1]) / jnp.maximum(c, 1.0)
        e_ref[...] = emb
        nrm = jnp.sqrt(jnp.sum(emb * emb, axis=1, keepdims=True))
        d_ref[...] = emb / nrm

    return pl.pallas_call(
        body,
        grid=(N // _RB,),
        in_specs=[pl.BlockSpec((NC, _RB, D), lambda i: (0, i, 0)),
                  pl.BlockSpec((NC, _RB, D), lambda i: (0, i, 0))],
        out_specs=[pl.BlockSpec((_RB, D), lambda i: (i, 0)),
                   pl.BlockSpec((_RB, D), lambda i: (i, 0))],
        out_shape=[jax.ShapeDtypeStruct((N, D), f32),
                   jax.ShapeDtypeStruct((N, D), f32)],
    )(p, cnt)


def _kmeanspp_body(x_ref, mu_ref, w_scr):
    xx = x_ref[...]
    c0 = xx[_I0:_I0 + 1, :]
    centers = [c0]
    closest = jnp.sum((xx - c0) ** 2, axis=1, keepdims=True)

    rowb = lax.broadcasted_iota(jnp.int32, (NB, N), 0)
    colb = lax.broadcasted_iota(jnp.int32, (NB, N), 1)
    B0 = (colb // RB2 == rowb).astype(f32)
    r50a = lax.broadcasted_iota(jnp.int32, (NB, NB), 0)
    r50b = lax.broadcasted_iota(jnp.int32, (NB, NB), 1)
    T50 = (r50b <= r50a).astype(f32)
    ra = lax.broadcasted_iota(jnp.int32, (RB2, RB2), 0)
    rb = lax.broadcasted_iota(jnp.int32, (RB2, RB2), 1)
    TR = (rb <= ra).astype(f32)
    oh_col = lax.broadcasted_iota(jnp.int32, (1, N), 1)

    for i in range(1, K):
        u = _US[i - 1]
        total = jnp.sum(closest)
        pos = total > 0.0
        w = jnp.where(pos, closest, 1.0)
        t = u * jnp.where(pos, total, float(N))
        sb = jnp.dot(B0, w, preferred_element_type=f32)
        bp = jnp.dot(T50, sb, preferred_element_type=f32)
        ble = bp <= t
        b_star = jnp.minimum(jnp.sum(ble.astype(jnp.int32)), NB - 1)
        off = jnp.sum(jnp.where(ble, sb, 0.0))
        w_scr[...] = w
        blk = w_scr[pl.ds(b_star * RB2, RB2), :]
        wp = jnp.dot(TR, blk, preferred_element_type=f32)
        cnt_in = jnp.sum((off + wp <= t).astype(jnp.int32))
        idx = jnp.minimum(b_star * RB2 + cnt_in, N - 1)
        oh = (oh_col == idx).astype(f32)
        ci = jnp.dot(oh, xx, preferred_element_type=f32)
        centers.append(ci)
        closest = jnp.minimum(closest,
                              jnp.sum((xx - ci) ** 2, axis=1, keepdims=True))
    mu_ref[...] = jnp.concatenate(centers, axis=0)


def _kmeanspp(data):
    return pl.pallas_call(
        _kmeanspp_body,
        out_shape=jax.ShapeDtypeStruct((K, D), f32),
        scratch_shapes=[pltpu.VMEM((N, 1), f32)],
    )(data)


def _cluster_body(ni_ref, x_ref, mu0_ref, mu_out, r_out, d_out):
    data = x_ref[...]
    ones_col = jnp.ones((N, 1), f32)

    def softmax_r(dist):
        sc = TEMP * dist
        m = jnp.max(sc, axis=1, keepdims=True)
        e = jnp.exp(sc - m)
        return e / jnp.sum(e, axis=1, keepdims=True)

    def body(_, mu):
        dist = lax.dot_general(data, mu, (((1,), (1,)), ((), ())),
                               preferred_element_type=f32)
        r = softmax_r(dist)
        cm = lax.dot_general(r, data, (((0,), (0,)), ((), ())),
                             preferred_element_type=f32)
        cr = lax.dot_general(r, ones_col, (((0,), (0,)), ((), ())),
                             preferred_element_type=f32)
        return cm / cr

    mu = lax.fori_loop(0, ni_ref[0], body, mu0_ref[...])
    dist = lax.dot_general(data, mu, (((1,), (1,)), ((), ())),
                           preferred_element_type=f32)
    r = softmax_r(dist)
    mu_out[...] = mu
    r_out[...] = r
    d_out[...] = dist


def _cluster(ni, data, mu0):
    return pl.pallas_call(
        _cluster_body,
        in_specs=[pl.BlockSpec(memory_space=pltpu.SMEM),
                  pl.BlockSpec((N, D), lambda: (0, 0)),
                  pl.BlockSpec((K, D), lambda: (0, 0))],
        out_specs=[pl.BlockSpec((K, D), lambda: (0, 0)),
                   pl.BlockSpec((N, K), lambda: (0, 0)),
                   pl.BlockSpec((N, K), lambda: (0, 0))],
        out_shape=[jax.ShapeDtypeStruct((K, D), f32),
                   jax.ShapeDtypeStruct((N, K), f32),
                   jax.ShapeDtypeStruct((N, K), f32)],
    )(ni, data, mu0)


def kernel(x, edge_index, W1, b1, W2, b2, num_iter):
    v_idx = edge_index[0]
    e_idx = edge_index[1]
    vi3 = v_idx.reshape(NW, NCH, CHUNK)
    ei3 = e_idx.reshape(NW, NCH, CHUNK)
    zrow = jnp.zeros((RPT, D), f32)
    ocnt = jnp.ones((CHUNK, D), f32)

    vcntp = _sc_count(vi3, ocnt, zrow)
    ecntp = _sc_count(ei3, ocnt, zrow)

    h1 = _mm_bias(x, W1, b1)
    pa = _sc_seg_sum(h1, vi3, ei3, zrow)
    he1 = _seg_mean(pa, ecntp)
    pb = _sc_seg_sum(he1, ei3, vi3, zrow)
    h2 = _seg_mean_relu_mm(pb, vcntp, W2, b2)
    pa2 = _sc_seg_sum(h2, vi3, ei3, zrow)
    he2 = _seg_mean(pa2, ecntp)
    pb2 = _sc_seg_sum(he2, ei3, vi3, zrow)
    embeds, data = _finalize(pb2, vcntp)

    mu0 = _kmeanspp(data)
    ni = jnp.asarray(num_iter, jnp.int32).reshape((1,))
    mu, r, dist = _cluster(ni, data, mu0)
    return mu, r, embeds, dist


# speedup vs baseline: 1.2487x; 1.0317x over previous
import functools

import numpy as np
import jax
import jax.numpy as jnp
from jax import lax
from jax.experimental import pallas as pl
from jax.experimental.pallas import tpu as pltpu
from jax.experimental.pallas import tpu_sc as plsc

N = 10000
INC = 320000
D = 128
K = 10
TEMP = 30.0

NC, NS = 2, 16
NW = NC * NS
PER_W = INC // NW
CHUNK = 125
NCH = PER_W // CHUNK
NP = 10112
RPT = NP // NS

f32 = jnp.float32


@functools.cache
def _sc_mesh():
    return plsc.VectorSubcoreMesh(core_axis_name="c", subcore_axis_name="s",
                                  num_cores=NC, num_subcores=NS)

_rng = np.random.RandomState(0)
_I0 = int(_rng.randint(N))
_US = [float(_rng.random_sample()) for _ in range(K - 1)]

NB = 50
RB2 = 200


@functools.cache
def _sc_seg_sum_kernel(width):
    def body(table, gidx, sidx, zeros, out,
             gi0, gi1, gi2, si0, si1, si2, r0, r1, r2,
             g0, g1, g2, s0, s1, s2, acc):
        c = lax.axis_index("c")
        s = lax.axis_index("s")
        wid = s * NC + c
        slot = [(gi0, si0, r0, g0, s0), (gi1, si1, r1, g1, s1),
                (gi2, si2, r2, g2, s2)]
        pltpu.sync_copy(zeros, acc.at[pl.ds(s * RPT, RPT)])
        plsc.subcore_barrier()

        def fire(j, p, wait_scatter):
            gi, si, rows, gsem, ssem = slot[p]
            if wait_scatter:
                pltpu.make_async_copy(rows, acc.at[si], ssem).wait()
            pltpu.sync_copy(gidx.at[wid, j], gi)
            pltpu.sync_copy(sidx.at[wid, j], si)
            pltpu.async_copy(table.at[gi], rows, gsem)

        def scat(j, p):
            gi, si, rows, gsem, ssem = slot[p]
            pltpu.make_async_copy(table.at[gi], rows, gsem).wait()
            pltpu.async_copy(rows, acc.at[si], ssem, add=True)

        fire(0, 0, False)
        fire(1, 1, False)

        def step(t, carry):
            for u in range(3):
                j = 3 * t + 2 + u
                p = (2 + u) % 3
                scat(j - 2, (p + 1) % 3)
                if u == 0:
                    @pl.when(t > 0)
                    def _():
                        gi, si, rows, gsem, ssem = slot[p]
                        pltpu.make_async_copy(rows, acc.at[si], ssem).wait()
                    gi, si, rows, gsem, ssem = slot[p]
                    pltpu.sync_copy(gidx.at[wid, j], gi)
                    pltpu.sync_copy(sidx.at[wid, j], si)
                    pltpu.async_copy(table.at[gi], rows, gsem)
                else:
                    fire(j, p, True)
            return carry

        lax.fori_loop(0, (NCH - 2) // 3, step, 0)
        scat(NCH - 2, (NCH - 2) % 3)
        scat(NCH - 1, (NCH - 1) % 3)
        for p in range(3):
            gi, si, rows, gsem, ssem = slot[p]
            pltpu.make_async_copy(rows, acc.at[si], ssem).wait()
        plsc.subcore_barrier()
        pltpu.sync_copy(acc.at[pl.ds(s * RPT, RPT)],
                        out.at[c, pl.ds(s * RPT, RPT)])

    idx_t = pltpu.VMEM((CHUNK,), jnp.int32)
    rows_t = pltpu.VMEM((CHUNK, width), f32)
    return pl.kernel(
        body,
        out_type=jax.ShapeDtypeStruct((NC, NP, width), f32),
        mesh=_sc_mesh(),
        scratch_types=[
            idx_t, idx_t, idx_t, idx_t, idx_t, idx_t,
            rows_t, rows_t, rows_t,
            pltpu.SemaphoreType.DMA, pltpu.SemaphoreType.DMA,
            pltpu.SemaphoreType.DMA, pltpu.SemaphoreType.DMA,
            pltpu.SemaphoreType.DMA, pltpu.SemaphoreType.DMA,
            pltpu.VMEM_SHARED((NP, width), f32),
        ],
    )


def _sc_seg_sum(table, gidx, sidx, zeros):
    return _sc_seg_sum_kernel(D)(table, gidx, sidx, zeros)


NCH2 = 2 * NCH


@functools.cache
def _sc_count_kernel():
    def body(vidx, eidx, ones_h, zeros, out, si_a, si_b, ones_v, acc):
        c = lax.axis_index("c")
        s = lax.axis_index("s")
        pltpu.sync_copy(zeros, acc.at[pl.ds(s * RPT, RPT)])
        pltpu.sync_copy(ones_h, ones_v)
        plsc.subcore_barrier()

        def make_loop(idx3):
            def step(t, carry):
                j = 2 * t
                pltpu.sync_copy(idx3.at[s, j], si_a)
                pltpu.sync_copy(ones_v, acc.at[si_a], add=True)
                pltpu.sync_copy(idx3.at[s, j + 1], si_b)
                pltpu.sync_copy(ones_v, acc.at[si_b], add=True)
                return carry
            return step

        @pl.when(c == 0)
        def _():
            lax.fori_loop(0, NCH2 // 2, make_loop(vidx), 0)

        @pl.when(c == 1)
        def _():
            lax.fori_loop(0, NCH2 // 2, make_loop(eidx), 0)

        plsc.subcore_barrier()
        pltpu.sync_copy(acc.at[pl.ds(s * RPT, RPT)],
                        out.at[c, pl.ds(s * RPT, RPT)])

    return pl.kernel(
        body,
        out_type=jax.ShapeDtypeStruct((NC, NP, D), f32),
        mesh=_sc_mesh(),
        scratch_types=[
            pltpu.VMEM((CHUNK,), jnp.int32),
            pltpu.VMEM((CHUNK,), jnp.int32),
            pltpu.VMEM((CHUNK, D), f32),
            pltpu.VMEM_SHARED((NP, D), f32),
        ],
    )


def _sc_counts2(vidx4, eidx4, ones_h, zeros):
    return _sc_count_kernel()(vidx4, eidx4, ones_h, zeros)


_RB = 2000


def _mm_bias_body(x_ref, w_ref, b_ref, o_ref):
    o_ref[...] = jnp.dot(x_ref[...], w_ref[...],
                         preferred_element_type=f32) + b_ref[...]


def _mm_bias(x, W, b):
    return pl.pallas_call(
        _mm_bias_body,
        grid=(N // _RB,),
        in_specs=[pl.BlockSpec((_RB, D), lambda i: (i, 0)),
                  pl.BlockSpec((D, D), lambda i: (0, 0)),
                  pl.BlockSpec((1, D), lambda i: (0, 0))],
        out_specs=pl.BlockSpec((_RB, D), lambda i: (i, 0)),
        out_shape=jax.ShapeDtypeStruct((N, D), f32),
    )(x, W, b.reshape(1, D))


def _seg_mean(p, cnt):
    def body(p_ref, c_ref, o_ref):
        c = c_ref[:, 0:1]
        o_ref[...] = (p_ref[0] + p_ref[1]) / jnp.maximum(c, 1.0)

    return pl.pallas_call(
        body,
        grid=(N // _RB,),
        in_specs=[pl.BlockSpec((NC, _RB, D), lambda i: (0, i, 0)),
                  pl.BlockSpec((_RB, D), lambda i: (i, 0))],
        out_specs=pl.BlockSpec((_RB, D), lambda i: (i, 0)),
        out_shape=jax.ShapeDtypeStruct((N, D), f32),
    )(p, cnt)


def _seg_mean_relu_mm(p, cnt, W, b):
    def body(p_ref, c_ref, w_ref, b_ref, o_ref):
        c = c_ref[:, 0:1]
        h = jnp.maximum((p_ref[0] + p_ref[1]) / jnp.maximum(c, 1.0), 0.0)
        o_ref[...] = jnp.dot(h, w_ref[...],
                             preferred_element_type=f32) + b_ref[...]

    return pl.pallas_call(
        body,
        grid=(N // _RB,),
        in_specs=[pl.BlockSpec((NC, _RB, D), lambda i: (0, i, 0)),
                  pl.BlockSpec((_RB, D), lambda i: (i, 0)),
                  pl.BlockSpec((D, D), lambda i: (0, 0)),
                  pl.BlockSpec((1, D), lambda i: (0, 0))],
        out_specs=pl.BlockSpec((_RB, D), lambda i: (i, 0)),
        out_shape=jax.ShapeDtypeStruct((N, D), f32),
    )(p, cnt, W, b.reshape(1, D))


def _finalize(p, cnt):
    def body(p_ref, c_ref, e_ref, d_ref):
        c = c_ref[:, 0:1]
        emb = (p_ref[0] + p_ref[1]) / jnp.maximum(c, 1.0)
        e_ref[...] = emb
        nrm = jnp.sqrt(jnp.sum(emb * emb, axis=1, keepdims=True))
        d_ref[...] = emb / nrm

    return pl.pallas_call(
        body,
        grid=(N // _RB,),
        in_specs=[pl.BlockSpec((NC, _RB, D), lambda i: (0, i, 0)),
                  pl.BlockSpec((_RB, D), lambda i: (i, 0))],
        out_specs=[pl.BlockSpec((_RB, D), lambda i: (i, 0)),
                   pl.BlockSpec((_RB, D), lambda i: (i, 0))],
        out_shape=[jax.ShapeDtypeStruct((N, D), f32),
                   jax.ShapeDtypeStruct((N, D), f32)],
    )(p, cnt)


def _kmeanspp_body(x_ref, mu_ref, w_scr):
    xx = x_ref[...]
    c0 = xx[_I0:_I0 + 1, :]
    centers = [c0]
    closest = jnp.sum((xx - c0) ** 2, axis=1, keepdims=True)

    rowb = lax.broadcasted_iota(jnp.int32, (NB, N), 0)
    colb = lax.broadcasted_iota(jnp.int32, (NB, N), 1)
    B0 = (colb // RB2 == rowb).astype(f32)
    r50a = lax.broadcasted_iota(jnp.int32, (NB, NB), 0)
    r50b = lax.broadcasted_iota(jnp.int32, (NB, NB), 1)
    T50 = (r50b <= r50a).astype(f32)
    ra = lax.broadcasted_iota(jnp.int32, (RB2, RB2), 0)
    rb = lax.broadcasted_iota(jnp.int32, (RB2, RB2), 1)
    TR = (rb <= ra).astype(f32)
    oh_col = lax.broadcasted_iota(jnp.int32, (1, N), 1)

    for i in range(1, K):
        u = _US[i - 1]
        total = jnp.sum(closest)
        pos = total > 0.0
        w = jnp.where(pos, closest, 1.0)
        t = u * jnp.where(pos, total, float(N))
        sb = jnp.dot(B0, w, preferred_element_type=f32)
        bp = jnp.dot(T50, sb, preferred_element_type=f32)
        ble = bp <= t
        b_star = jnp.minimum(jnp.sum(ble.astype(jnp.int32)), NB - 1)
        off = jnp.sum(jnp.where(ble, sb, 0.0))
        w_scr[...] = w
        blk = w_scr[pl.ds(b_star * RB2, RB2), :]
        wp = jnp.dot(TR, blk, preferred_element_type=f32)
        cnt_in = jnp.sum((off + wp <= t).astype(jnp.int32))
        idx = jnp.minimum(b_star * RB2 + cnt_in, N - 1)
        oh = (oh_col == idx).astype(f32)
        ci = jnp.dot(oh, xx, preferred_element_type=f32)
        centers.append(ci)
        closest = jnp.minimum(closest,
                              jnp.sum((xx - ci) ** 2, axis=1, keepdims=True))
    mu_ref[...] = jnp.concatenate(centers, axis=0)


def _kmeanspp(data):
    return pl.pallas_call(
        _kmeanspp_body,
        out_shape=jax.ShapeDtypeStruct((K, D), f32),
        scratch_shapes=[pltpu.VMEM((N, 1), f32)],
    )(data)


def _cluster_body(ni_ref, x_ref, mu0_ref, mu_out, r_out, d_out):
    data = x_ref[...]
    ones_col = jnp.ones((N, 1), f32)

    def softmax_r(dist):
        sc = TEMP * dist
        m = jnp.max(sc, axis=1, keepdims=True)
        e = jnp.exp(sc - m)
        return e / jnp.sum(e, axis=1, keepdims=True)

    def body(_, mu):
        dist = lax.dot_general(data, mu, (((1,), (1,)), ((), ())),
                               preferred_element_type=f32)
        r = softmax_r(dist)
        cm = lax.dot_general(r, data, (((0,), (0,)), ((), ())),
                             preferred_element_type=f32)
        cr = lax.dot_general(r, ones_col, (((0,), (0,)), ((), ())),
                             preferred_element_type=f32)
        return cm / cr

    mu = lax.fori_loop(0, ni_ref[0], body, mu0_ref[...])
    dist = lax.dot_general(data, mu, (((1,), (1,)), ((), ())),
                           preferred_element_type=f32)
    r = softmax_r(dist)
    mu_out[...] = mu
    r_out[...] = r
    d_out[...] = dist


def _cluster(ni, data, mu0):
    return pl.pallas_call(
        _cluster_body,
        in_specs=[pl.BlockSpec(memory_space=pltpu.SMEM),
                  pl.BlockSpec((N, D), lambda: (0, 0)),
                  pl.BlockSpec((K, D), lambda: (0, 0))],
        out_specs=[pl.BlockSpec((K, D), lambda: (0, 0)),
                   pl.BlockSpec((N, K), lambda: (0, 0)),
                   pl.BlockSpec((N, K), lambda: (0, 0))],
        out_shape=[jax.ShapeDtypeStruct((K, D), f32),
                   jax.ShapeDtypeStruct((N, K), f32),
                   jax.ShapeDtypeStruct((N, K), f32)],
    )(ni, data, mu0)


def kernel(x, edge_index, W1, b1, W2, b2, num_iter):
    v_idx = edge_index[0]
    e_idx = edge_index[1]
    vi3 = v_idx.reshape(NW, NCH, CHUNK)
    ei3 = e_idx.reshape(NW, NCH, CHUNK)
    vi4 = v_idx.reshape(NS, NCH2, CHUNK)
    ei4 = e_idx.reshape(NS, NCH2, CHUNK)
    zrow = jnp.zeros((RPT, D), f32)
    ocnt = jnp.ones((CHUNK, D), f32)

    cnts = _sc_counts2(vi4, ei4, ocnt, zrow)
    vcntp = cnts[0]
    ecntp = cnts[1]

    h1 = _mm_bias(x, W1, b1)
    pa = _sc_seg_sum(h1, vi3, ei3, zrow)
    he1 = _seg_mean(pa, ecntp)
    pb = _sc_seg_sum(he1, ei3, vi3, zrow)
    h2 = _seg_mean_relu_mm(pb, vcntp, W2, b2)
    pa2 = _sc_seg_sum(h2, vi3, ei3, zrow)
    he2 = _seg_mean(pa2, ecntp)
    pb2 = _sc_seg_sum(he2, ei3, vi3, zrow)
    embeds, data = _finalize(pb2, vcntp)

    mu0 = _kmeanspp(data)
    ni = jnp.asarray(num_iter, jnp.int32).reshape((1,))
    mu, r, dist = _cluster(ni, data, mu0)
    return mu, r, embeds, dist
